# Initial kernel scaffold; baseline (speedup 1.0000x reference)
import jax
import jax.numpy as jnp
from jax.experimental import pallas as pl


def kernel(x, edge_index, W1, b1, Wh, bh, Wo, bo):
    raise NotImplementedError("write your pallas kernel here")


# speedup vs baseline: 10.1103x; 10.1103x over previous
import functools

import jax
import jax.numpy as jnp
from jax import lax
from jax.experimental import pallas as pl
from jax.experimental.pallas import tpu as pltpu
from jax.experimental.pallas import tpu_sc as plsc

N_NODES = 10000
D = 128
N_EDGES = 320000

CHUNK = 128
CHUNKS = 158
N_CHUNKS = 16 * CHUNKS
E_PAD = N_CHUNKS * CHUNK
N_PAD = 10240
ROWS_PER_TILE = N_PAD // 16
HALF_CHUNKS = CHUNKS // 2

_mesh = plsc.VectorSubcoreMesh(core_axis_name="c", subcore_axis_name="s")
_f32 = jnp.float32
_sc_params = pltpu.CompilerParams(use_tc_tiling_on_sc=False)


@functools.partial(
    pl.kernel,
    out_type=jax.ShapeDtypeStruct((2, N_PAD, D), _f32),
    mesh=_mesh,
    scratch_types=[
        pltpu.VMEM((CHUNK,), jnp.int32),
        pltpu.VMEM((CHUNK, D), _f32),
        pltpu.VMEM((CHUNK, D), _f32),
        pltpu.VMEM_SHARED((N_PAD, D), _f32),
        pltpu.SemaphoreType.DMA,
    ],
    compiler_params=_sc_params,
)
def _deg_kernel(col_hbm, ones_hbm, zeros_hbm, deg_out, idx_v, ones_v, zero_v,
                acc_sh, sem):
    cid = lax.axis_index("c")
    sid = lax.axis_index("s")
    base = sid * ROWS_PER_TILE

    pltpu.sync_copy(ones_hbm, ones_v)
    pltpu.sync_copy(zeros_hbm, zero_v)
    for k in range(ROWS_PER_TILE // CHUNK):
        pltpu.sync_copy(zero_v, acc_sh.at[pl.ds(base + k * CHUNK, CHUNK)])
    plsc.subcore_barrier()

    def body(j, carry):
        g = sid * CHUNKS + 2 * j + cid
        pltpu.sync_copy(col_hbm.at[g], idx_v)
        pltpu.sync_copy(ones_v, acc_sh.at[idx_v], add=True)
        return carry

    lax.fori_loop(0, HALF_CHUNKS, body, 0)
    plsc.subcore_barrier()

    for k in range(ROWS_PER_TILE // CHUNK):
        pltpu.sync_copy(acc_sh.at[pl.ds(base + k * CHUNK, CHUNK)], zero_v)
        pltpu.sync_copy(zero_v, deg_out.at[cid, pl.ds(base + k * CHUNK, CHUNK)])


@functools.partial(
    pl.kernel,
    out_type=jax.ShapeDtypeStruct((2, N_PAD, D), _f32),
    mesh=_mesh,
    scratch_types=[
        pltpu.VMEM((CHUNK,), jnp.int32),
        pltpu.VMEM((CHUNK,), jnp.int32),
        pltpu.VMEM((CHUNK, D), _f32),
        pltpu.VMEM((CHUNK, D), _f32),
        pltpu.VMEM_SHARED((N_PAD, D), _f32),
        pltpu.SemaphoreType.DMA,
    ],
    compiler_params=_sc_params,
)
def _spmm_kernel(y_hbm, src_hbm, dst_hbm, zeros_hbm, s_out, gidx_v, sidx_v,
                 gb, zero_v, acc_sh, sem):
    cid = lax.axis_index("c")
    sid = lax.axis_index("s")
    base = sid * ROWS_PER_TILE

    pltpu.sync_copy(zeros_hbm, zero_v)
    for k in range(ROWS_PER_TILE // CHUNK):
        pltpu.sync_copy(zero_v, acc_sh.at[pl.ds(base + k * CHUNK, CHUNK)])
    plsc.subcore_barrier()

    def body(j, carry):
        g = sid * CHUNKS + 2 * j + cid
        pltpu.sync_copy(src_hbm.at[g], gidx_v)
        pltpu.sync_copy(dst_hbm.at[g], sidx_v)
        pltpu.async_copy(y_hbm.at[gidx_v], gb, sem).wait()
        pltpu.sync_copy(gb, acc_sh.at[sidx_v], add=True)
        return carry

    lax.fori_loop(0, HALF_CHUNKS, body, 0)
    plsc.subcore_barrier()

    for k in range(ROWS_PER_TILE // CHUNK):
        pltpu.sync_copy(acc_sh.at[pl.ds(base + k * CHUNK, CHUNK)], zero_v)
        pltpu.sync_copy(zero_v, s_out.at[cid, pl.ds(base + k * CHUNK, CHUNK)])


def _tc_prep_body(deg_ref, x_ref, w1_ref, d_ref, y_ref, dw_ref):
    d = lax.rsqrt(deg_ref[...] + 1.0)
    d_ref[...] = d
    xl = jnp.dot(x_ref[...], w1_ref[...], preferred_element_type=_f32)
    zpad = jnp.zeros((N_PAD - N_NODES, D), _f32)
    y_ref[0:N_NODES, :] = d[0:N_NODES] * xl
    y_ref[N_NODES:N_PAD, :] = zpad
    dw_ref[...] = jnp.broadcast_to(d, (N_PAD, D))


def _tc_mid_body(s_ref, y_ref, d_ref, b_ref, w_ref, out_ref):
    d = d_ref[0:N_NODES]
    s = s_ref[0, 0:N_NODES, :] + s_ref[1, 0:N_NODES, :] + y_ref[0:N_NODES, :]
    h = jnp.maximum(d * s + b_ref[...], 0.0)
    xl = jnp.dot(h, w_ref[...], preferred_element_type=_f32)
    out_ref[0:N_NODES, :] = d * xl
    out_ref[N_NODES:N_PAD, :] = jnp.zeros((N_PAD - N_NODES, D), _f32)


def _tc_final_body(s_ref, y_ref, d_ref, b_ref, c_ref, wo_ref, bo_ref,
                   out_ref):
    d = d_ref[0:N_NODES]
    s = s_ref[0, 0:N_NODES, :] + s_ref[1, 0:N_NODES, :] + y_ref[0:N_NODES, :]
    h2 = jnp.maximum(d * s + b_ref[...], 0.0)
    w = d * (c_ref[0:N_NODES] + d)
    z = jnp.sum(h2 * w, axis=0, keepdims=True)
    out_ref[...] = (jnp.dot(z, wo_ref[...], preferred_element_type=_f32)
                    * (1.0 / N_NODES) + bo_ref[...])


def kernel(x, edge_index, W1, b1, Wh, bh, Wo, bo):
    ei = edge_index.astype(jnp.int32)
    pad = jnp.full((E_PAD - N_EDGES,), N_NODES, jnp.int32)
    row2 = jnp.concatenate([ei[0], pad]).reshape(N_CHUNKS, CHUNK)
    col2 = jnp.concatenate([ei[1], pad]).reshape(N_CHUNKS, CHUNK)

    ones_t = jnp.ones((CHUNK, D), _f32)
    zeros_t = jnp.zeros((CHUNK, D), _f32)

    degw = _deg_kernel(col2, ones_t, zeros_t)
    deg = degw[0, :, 0:1] + degw[1, :, 0:1]

    d_arr, y1, dwide = pl.pallas_call(
        _tc_prep_body,
        out_shape=(jax.ShapeDtypeStruct((N_PAD, 1), _f32),
                   jax.ShapeDtypeStruct((N_PAD, D), _f32),
                   jax.ShapeDtypeStruct((N_PAD, D), _f32)),
    )(deg, x, W1)

    s1 = _spmm_kernel(y1, row2, col2, zeros_t)
    cw = _spmm_kernel(dwide, col2, row2, zeros_t)
    c_arr = cw[0, :, 0:1] + cw[1, :, 0:1]

    y2 = pl.pallas_call(
        _tc_mid_body,
        out_shape=jax.ShapeDtypeStruct((N_PAD, D), _f32),
    )(s1, y1, d_arr, b1.reshape(1, D), Wh)

    s2 = _spmm_kernel(y2, row2, col2, zeros_t)

    out = pl.pallas_call(
        _tc_final_body,
        out_shape=jax.ShapeDtypeStruct((1, D), _f32),
    )(s2, y2, d_arr, bh.reshape(1, D), c_arr, Wo, bo.reshape(1, D))
    return out


# speedup vs baseline: 12.4732x; 1.2337x over previous
import functools

import jax
import jax.numpy as jnp
from jax import lax
from jax.experimental import pallas as pl
from jax.experimental.pallas import tpu as pltpu
from jax.experimental.pallas import tpu_sc as plsc

N_NODES = 10000
D = 128
N_EDGES = 320000

CHUNK = 128
CHUNKS = 158
N_CHUNKS = 16 * CHUNKS
E_PAD = N_CHUNKS * CHUNK
N_PAD = 10240
ROWS_PER_TILE = N_PAD // 16
HALF_CHUNKS = CHUNKS // 2

_mesh = plsc.VectorSubcoreMesh(core_axis_name="c", subcore_axis_name="s")
_f32 = jnp.float32
_sc_params = pltpu.CompilerParams(use_tc_tiling_on_sc=False)


@functools.partial(
    pl.kernel,
    out_type=jax.ShapeDtypeStruct((2, N_PAD, D), _f32),
    mesh=_mesh,
    scratch_types=[
        pltpu.VMEM((CHUNK,), jnp.int32),
        pltpu.VMEM((CHUNK, D), _f32),
        pltpu.VMEM((CHUNK, D), _f32),
        pltpu.VMEM_SHARED((N_PAD, D), _f32),
        pltpu.SemaphoreType.DMA,
    ],
    compiler_params=_sc_params,
)
def _deg_kernel(col_hbm, ones_hbm, zeros_hbm, deg_out, idx_v, ones_v, zero_v,
                acc_sh, sem):
    cid = lax.axis_index("c")
    sid = lax.axis_index("s")
    base = sid * ROWS_PER_TILE

    pltpu.sync_copy(ones_hbm, ones_v)
    pltpu.sync_copy(zeros_hbm, zero_v)
    for k in range(ROWS_PER_TILE // CHUNK):
        pltpu.sync_copy(zero_v, acc_sh.at[pl.ds(base + k * CHUNK, CHUNK)])
    plsc.subcore_barrier()

    def body(j, carry):
        g = sid * CHUNKS + 2 * j + cid
        pltpu.sync_copy(col_hbm.at[g], idx_v)
        pltpu.sync_copy(ones_v, acc_sh.at[idx_v], add=True)
        return carry

    lax.fori_loop(0, HALF_CHUNKS, body, 0)
    plsc.subcore_barrier()

    for k in range(ROWS_PER_TILE // CHUNK):
        pltpu.sync_copy(acc_sh.at[pl.ds(base + k * CHUNK, CHUNK)], zero_v)
        pltpu.sync_copy(zero_v, deg_out.at[cid, pl.ds(base + k * CHUNK, CHUNK)])


@functools.partial(
    pl.kernel,
    out_type=jax.ShapeDtypeStruct((2, N_PAD, D), _f32),
    mesh=_mesh,
    scratch_types=[
        pltpu.VMEM((CHUNK,), jnp.int32),
        pltpu.VMEM((CHUNK,), jnp.int32),
        pltpu.VMEM((CHUNK,), jnp.int32),
        pltpu.VMEM((CHUNK,), jnp.int32),
        pltpu.VMEM((CHUNK, D), _f32),
        pltpu.VMEM((CHUNK, D), _f32),
        pltpu.VMEM_SHARED((N_PAD, D), _f32),
        pltpu.SemaphoreType.DMA,
        pltpu.SemaphoreType.DMA,
    ],
    compiler_params=_sc_params,
)
def _spmm_kernel(y_hbm, src_hbm, dst_hbm, zeros_hbm, s_out, gidx_a, sidx_a,
                 gidx_b, sidx_b, gb_a, gb_b, acc_sh, sem_a, sem_b):
    cid = lax.axis_index("c")
    sid = lax.axis_index("s")
    base = sid * ROWS_PER_TILE
    g0 = sid * CHUNKS + cid

    pltpu.sync_copy(zeros_hbm, gb_a)
    for k in range(ROWS_PER_TILE // CHUNK):
        pltpu.sync_copy(gb_a, acc_sh.at[pl.ds(base + k * CHUNK, CHUNK)])
    plsc.subcore_barrier()

    pltpu.sync_copy(src_hbm.at[g0], gidx_a)
    pltpu.sync_copy(dst_hbm.at[g0], sidx_a)
    pltpu.async_copy(y_hbm.at[gidx_a], gb_a, sem_a)

    def wait_gather(gidx, gb, sem):
        pltpu.make_async_copy(y_hbm.at[gidx], gb, sem).wait()

    def body(jj, carry):
        g = g0 + 4 * jj
        pltpu.sync_copy(src_hbm.at[g + 2], gidx_b)
        pltpu.sync_copy(dst_hbm.at[g + 2], sidx_b)
        pltpu.async_copy(y_hbm.at[gidx_b], gb_b, sem_b)
        wait_gather(gidx_a, gb_a, sem_a)
        pltpu.sync_copy(gb_a, acc_sh.at[sidx_a], add=True)
        pltpu.sync_copy(src_hbm.at[g + 4], gidx_a)
        pltpu.sync_copy(dst_hbm.at[g + 4], sidx_a)
        pltpu.async_copy(y_hbm.at[gidx_a], gb_a, sem_a)
        wait_gather(gidx_b, gb_b, sem_b)
        pltpu.sync_copy(gb_b, acc_sh.at[sidx_b], add=True)
        return carry

    lax.fori_loop(0, (HALF_CHUNKS - 1) // 2, body, 0)
    wait_gather(gidx_a, gb_a, sem_a)
    pltpu.sync_copy(gb_a, acc_sh.at[sidx_a], add=True)
    plsc.subcore_barrier()

    for k in range(ROWS_PER_TILE // CHUNK):
        pltpu.sync_copy(acc_sh.at[pl.ds(base + k * CHUNK, CHUNK)], gb_a)
        pltpu.sync_copy(gb_a, s_out.at[cid, pl.ds(base + k * CHUNK, CHUNK)])


def _tc_prep_body(deg_ref, x_ref, w1_ref, d_ref, y_ref, dw_ref):
    d = lax.rsqrt(deg_ref[...] + 1.0)
    d_ref[...] = d
    xl = jnp.dot(x_ref[...], w1_ref[...], preferred_element_type=_f32)
    zpad = jnp.zeros((N_PAD - N_NODES, D), _f32)
    y_ref[0:N_NODES, :] = d[0:N_NODES] * xl
    y_ref[N_NODES:N_PAD, :] = zpad
    dw_ref[...] = jnp.broadcast_to(d, (N_PAD, D))


def _tc_mid_body(s_ref, y_ref, d_ref, b_ref, w_ref, out_ref):
    d = d_ref[0:N_NODES]
    s = s_ref[0, 0:N_NODES, :] + s_ref[1, 0:N_NODES, :] + y_ref[0:N_NODES, :]
    h = jnp.maximum(d * s + b_ref[...], 0.0)
    xl = jnp.dot(h, w_ref[...], preferred_element_type=_f32)
    out_ref[0:N_NODES, :] = d * xl
    out_ref[N_NODES:N_PAD, :] = jnp.zeros((N_PAD - N_NODES, D), _f32)


def _tc_final_body(s_ref, y_ref, d_ref, b_ref, c_ref, wo_ref, bo_ref,
                   out_ref):
    d = d_ref[0:N_NODES]
    s = s_ref[0, 0:N_NODES, :] + s_ref[1, 0:N_NODES, :] + y_ref[0:N_NODES, :]
    h2 = jnp.maximum(d * s + b_ref[...], 0.0)
    w = d * (c_ref[0:N_NODES] + d)
    z = jnp.sum(h2 * w, axis=0, keepdims=True)
    out_ref[...] = (jnp.dot(z, wo_ref[...], preferred_element_type=_f32)
                    * (1.0 / N_NODES) + bo_ref[...])


def kernel(x, edge_index, W1, b1, Wh, bh, Wo, bo):
    ei = edge_index.astype(jnp.int32)
    pad = jnp.full((E_PAD - N_EDGES,), N_NODES, jnp.int32)
    row2 = jnp.concatenate([ei[0], pad]).reshape(N_CHUNKS, CHUNK)
    col2 = jnp.concatenate([ei[1], pad]).reshape(N_CHUNKS, CHUNK)

    ones_t = jnp.ones((CHUNK, D), _f32)
    zeros_t = jnp.zeros((CHUNK, D), _f32)

    degw = _deg_kernel(col2, ones_t, zeros_t)
    deg = degw[0, :, 0:1] + degw[1, :, 0:1]

    d_arr, y1, dwide = pl.pallas_call(
        _tc_prep_body,
        out_shape=(jax.ShapeDtypeStruct((N_PAD, 1), _f32),
                   jax.ShapeDtypeStruct((N_PAD, D), _f32),
                   jax.ShapeDtypeStruct((N_PAD, D), _f32)),
    )(deg, x, W1)

    s1 = _spmm_kernel(y1, row2, col2, zeros_t)
    cw = _spmm_kernel(dwide, col2, row2, zeros_t)
    c_arr = cw[0, :, 0:1] + cw[1, :, 0:1]

    y2 = pl.pallas_call(
        _tc_mid_body,
        out_shape=jax.ShapeDtypeStruct((N_PAD, D), _f32),
    )(s1, y1, d_arr, b1.reshape(1, D), Wh)

    s2 = _spmm_kernel(y2, row2, col2, zeros_t)

    out = pl.pallas_call(
        _tc_final_body,
        out_shape=jax.ShapeDtypeStruct((1, D), _f32),
    )(s2, y2, d_arr, bh.reshape(1, D), c_arr, Wo, bo.reshape(1, D))
    return out
